# baseline (speedup 1.0000x reference)
import functools

import jax
import jax.numpy as jnp
from jax.experimental import pallas as pl
from jax.experimental.pallas import tpu as pltpu

_SPLITS = 8


def _wls_body(*refs, inv_m):
    adj_refs = refs[:_SPLITS]
    feat_ref = refs[_SPLITS]
    o_ref = refs[_SPLITS + 1]
    adj_sum = adj_refs[0][...].sum(axis=1)
    for r in adj_refs[1:]:
        adj_sum += r[...].sum(axis=1)
    feat = feat_ref[...]
    a16 = adj_sum.astype(jnp.bfloat16)
    f16 = (feat * inv_m).astype(jnp.bfloat16)
    agg = jax.lax.dot_general(
        a16, f16,
        dimension_numbers=(((2,), (1,)), ((0,), (0,))),
        preferred_element_type=jnp.float32,
    )
    o_ref[...] = feat + agg


def kernel(node_feat, adj):
    B, N, D = node_feat.shape
    _, M, _, _ = adj.shape
    inv_m = 1.0 / float(M)

    block_b = 8
    block_m = M // _SPLITS
    grid = (B // block_b,)
    adj_specs = [
        pl.BlockSpec((block_b, block_m, N, N), lambda b, k=k: (b, k, 0, 0))
        for k in range(_SPLITS)
    ]
    return pl.pallas_call(
        functools.partial(_wls_body, inv_m=inv_m),
        out_shape=jax.ShapeDtypeStruct((B, N, D), node_feat.dtype),
        grid=grid,
        in_specs=adj_specs + [pl.BlockSpec((block_b, N, D), lambda b: (b, 0, 0))],
        out_specs=pl.BlockSpec((block_b, N, D), lambda b: (b, 0, 0)),
        compiler_params=pltpu.CompilerParams(
            dimension_semantics=("parallel",),
            vmem_limit_bytes=64 * 1024 * 1024,
        ),
    )(*([adj] * _SPLITS), node_feat)


# speedup vs baseline: 1.0080x; 1.0080x over previous
import functools

import jax
import jax.numpy as jnp
from jax.experimental import pallas as pl
from jax.experimental.pallas import tpu as pltpu


def _wls_body(adj_ref, feat_ref, o_ref, *, inv_m):
    adj_sum = jnp.sum(adj_ref[...], axis=1)
    feat = feat_ref[...]
    a16 = adj_sum.astype(jnp.bfloat16)
    f16 = (feat * inv_m).astype(jnp.bfloat16)
    agg = jax.lax.dot_general(
        a16, f16,
        dimension_numbers=(((2,), (1,)), ((0,), (0,))),
        preferred_element_type=jnp.float32,
    )
    o_ref[...] = feat + agg


def kernel(node_feat, adj):
    B, N, D = node_feat.shape
    _, M, _, _ = adj.shape
    inv_m = 1.0 / float(M)

    block_b = 8
    while B % block_b != 0:
        block_b -= 1
    grid = (B // block_b,)
    return pl.pallas_call(
        functools.partial(_wls_body, inv_m=inv_m),
        out_shape=jax.ShapeDtypeStruct((B, N, D), node_feat.dtype),
        grid=grid,
        in_specs=[
            pl.BlockSpec((block_b, M, N, N), lambda b: (b, 0, 0, 0)),
            pl.BlockSpec((block_b, N, D), lambda b: (b, 0, 0)),
        ],
        out_specs=pl.BlockSpec((block_b, N, D), lambda b: (b, 0, 0)),
        compiler_params=pltpu.CompilerParams(
            dimension_semantics=("parallel",),
            vmem_limit_bytes=64 * 1024 * 1024,
        ),
    )(adj, node_feat)
